# baseline (speedup 1.0000x reference)
import numpy as np
import jax
import jax.numpy as jnp
from jax.experimental import pallas as pl
from jax.experimental.pallas import tpu as pltpu

_N = 64
_EU = 12
_E = 690
_ES = 768
_BB = 16


def _edge_tables():
    src, dst = [], []
    for i in range(_N):
        for j in range(max(0, i - _EU), i):
            src.append(j)
            dst.append(i)
    src = np.asarray(src, np.int32)
    dst = np.asarray(dst, np.int32)
    bandsel = np.zeros((_N * _N, _ES), np.float32)
    bandsel[dst * _N + src, np.arange(_E)] = 1.0
    ppair = np.zeros((_ES, _N), np.float32)
    ppair[np.arange(_E), dst] = 1.0
    ppair[np.arange(_E), src] += 1.0
    return bandsel, ppair


_BANDSEL, _PPAIR = _edge_tables()


def _body(x_ref, adj_ref, w0_ref, w1_ref, w2_ref, wp_ref, wn_ref,
          bn_ref, we_ref, be_ref, bs_ref, pp_ref, zn_ref, ze_ref):
    f32 = jnp.float32
    x2d = x_ref[...].reshape(_BB * _N, 16)
    adj = adj_ref[...]

    h = x2d
    for wref in (w0_ref, w1_ref, w2_ref):
        mm = jnp.dot(h, wref[...], preferred_element_type=f32)
        accs = []
        for b in range(_BB):
            adj_cat = jnp.concatenate([adj[b, r] for r in range(4)], axis=1)
            m_cat = jnp.concatenate(
                [mm[b * _N:(b + 1) * _N, 128 * r:128 * (r + 1)] for r in range(4)],
                axis=0)
            accs.append(jnp.dot(adj_cat, m_cat, preferred_element_type=f32))
        h = jax.nn.relu(jnp.concatenate(accs, axis=0))

    h = jnp.tanh(jnp.dot(h, wp_ref[...], preferred_element_type=f32))

    stn = jnp.dot(h, wn_ref[...], preferred_element_type=f32) + bn_ref[...]
    zn2d = x2d * jax.nn.sigmoid(stn[:, :16] + 2.0) + stn[:, 16:32]
    zn_ref[...] = zn2d.reshape(_BB, _N, 16)

    h_cat = jnp.concatenate([h[b * _N:(b + 1) * _N, :] for b in range(_BB)],
                            axis=1)
    he_sel = jnp.dot(h_cat, we_ref[...], preferred_element_type=f32)
    st = jnp.dot(pp_ref[...], he_sel, preferred_element_type=f32) + be_ref[...]

    adj_flat = adj.reshape(_BB * 4, _N, _N).reshape(_BB * 4, _N * _N)
    band_t = jnp.dot(adj_flat, bs_ref[...], preferred_element_type=f32)
    band = jnp.transpose(band_t)

    ze = band * jax.nn.sigmoid(st[:, :4 * _BB] + 2.0) + st[:, 4 * _BB:]
    for b in range(_BB):
        ze_ref[b] = ze[:_E, 4 * b:4 * (b + 1)]


def _flow(x, adj, w0c, w1c, w2c, wp, wn, bn, we, becat, bs, pp,
          *, interpret=False):
    b = x.shape[0]
    grid = (b // _BB,)
    full = lambda i: (0, 0)
    zn, ze = pl.pallas_call(
        _body,
        grid=grid,
        in_specs=[
            pl.BlockSpec((_BB, _N, 16), lambda i: (i, 0, 0)),
            pl.BlockSpec((_BB, 4, _N, _N), lambda i: (i, 0, 0, 0)),
            pl.BlockSpec((16, 512), full),
            pl.BlockSpec((128, 512), full),
            pl.BlockSpec((128, 512), full),
            pl.BlockSpec((128, 128), full),
            pl.BlockSpec((128, 32), full),
            pl.BlockSpec((1, 32), full),
            pl.BlockSpec((128 * _BB, 8 * _BB), full),
            pl.BlockSpec((1, 8 * _BB), full),
            pl.BlockSpec((_N * _N, _ES), full),
            pl.BlockSpec((_ES, _N), full),
        ],
        out_specs=[
            pl.BlockSpec((_BB, _N, 16), lambda i: (i, 0, 0)),
            pl.BlockSpec((_BB, _E, 4), lambda i: (i, 0, 0)),
        ],
        out_shape=[
            jax.ShapeDtypeStruct((b, _N, 16), jnp.float32),
            jax.ShapeDtypeStruct((b, _E, 4), jnp.float32),
        ],
        compiler_params=pltpu.CompilerParams(
            dimension_semantics=("parallel",)),
        interpret=interpret,
    )(x, adj, w0c, w1c, w2c, wp, wn, bn, we, becat, bs, pp)
    return zn, ze


def kernel(inp_node_features, inp_adj_features, W0, W1, W2, Wproj,
           Wst_node, bst_node, Wst_edge, bst_edge):
    b = inp_node_features.shape[0]
    w0c = jnp.concatenate([W0[r] for r in range(4)], axis=1)
    w1c = jnp.concatenate([W1[r] for r in range(4)], axis=1)
    w2c = jnp.concatenate([W2[r] for r in range(4)], axis=1)
    wblk = jnp.zeros((128 * _BB, 8 * _BB), jnp.float32)
    for bb in range(_BB):
        wblk = wblk.at[128 * bb:128 * (bb + 1), 4 * bb:4 * (bb + 1)].set(
            Wst_edge[:, :4])
        wblk = wblk.at[128 * bb:128 * (bb + 1),
                       4 * _BB + 4 * bb:4 * _BB + 4 * (bb + 1)].set(
            Wst_edge[:, 4:])
    becat = jnp.concatenate([jnp.tile(bst_edge[:4], _BB),
                             jnp.tile(bst_edge[4:], _BB)]).reshape(1, 8 * _BB)
    zn, zew = _flow(inp_node_features, inp_adj_features, w0c, w1c,
                    w2c, Wproj, Wst_node, bst_node.reshape(1, 32), wblk,
                    becat, jnp.asarray(_BANDSEL), jnp.asarray(_PPAIR))
    return jnp.concatenate([zn.reshape(b, -1), zew.reshape(b, -1)], axis=1)


# speedup vs baseline: 1.3063x; 1.3063x over previous
import numpy as np
import jax
import jax.numpy as jnp
from jax.experimental import pallas as pl
from jax.experimental.pallas import tpu as pltpu

_N = 64
_EU = 12
_E = 690
_ES = 768
_BB = 32


def _edge_tables():
    src, dst = [], []
    for i in range(_N):
        for j in range(max(0, i - _EU), i):
            src.append(j)
            dst.append(i)
    src = np.asarray(src, np.int32)
    dst = np.asarray(dst, np.int32)
    bandsel = np.zeros((_N * _N, _ES), np.float32)
    bandsel[dst * _N + src, np.arange(_E)] = 1.0
    ppair = np.zeros((_ES, _N), np.float32)
    ppair[np.arange(_E), dst] = 1.0
    ppair[np.arange(_E), src] += 1.0
    return bandsel, ppair


_BANDSEL, _PPAIR = _edge_tables()


def _body(x_ref, adj_ref, w0_ref, w1_ref, w2_ref, wp_ref, wn_ref,
          bn_ref, we_ref, be_ref, bs_ref, pp_ref, zn_ref, ze_ref):
    f32 = jnp.float32
    x2d = x_ref[...].reshape(_BB * _N, 16)
    adj = adj_ref[...]

    h = x2d
    for wref in (w0_ref, w1_ref, w2_ref):
        mm = jnp.dot(h, wref[...], preferred_element_type=f32)
        accs = []
        for b in range(_BB):
            adj_cat = jnp.concatenate([adj[b, r] for r in range(4)], axis=1)
            m_cat = jnp.concatenate(
                [mm[b * _N:(b + 1) * _N, 128 * r:128 * (r + 1)] for r in range(4)],
                axis=0)
            accs.append(jnp.dot(adj_cat, m_cat, preferred_element_type=f32))
        h = jax.nn.relu(jnp.concatenate(accs, axis=0))

    h = jnp.tanh(jnp.dot(h, wp_ref[...], preferred_element_type=f32))

    stn = jnp.dot(h, wn_ref[...], preferred_element_type=f32) + bn_ref[...]
    zn2d = x2d * jax.nn.sigmoid(stn[:, :16] + 2.0) + stn[:, 16:32]
    zn_ref[...] = zn2d.reshape(_BB, _N, 16)

    h_cat = jnp.concatenate([h[b * _N:(b + 1) * _N, :] for b in range(_BB)],
                            axis=1)
    he_sel = jnp.dot(h_cat, we_ref[...], preferred_element_type=f32)
    st = jnp.dot(pp_ref[...], he_sel, preferred_element_type=f32) + be_ref[...]

    adj_flat = adj.reshape(_BB * 4, _N, _N).reshape(_BB * 4, _N * _N)
    band_t = jnp.dot(adj_flat, bs_ref[...], preferred_element_type=f32)
    band = jnp.transpose(band_t)

    ze = band * jax.nn.sigmoid(st[:, :4 * _BB] + 2.0) + st[:, 4 * _BB:]
    ze_ref[0] = ze[:_E, :]


def _flow(x, adj, w0c, w1c, w2c, wp, wn, bn, we, becat, bs, pp,
          *, interpret=False):
    b = x.shape[0]
    grid = (b // _BB,)
    full = lambda i: (0, 0)
    zn, ze = pl.pallas_call(
        _body,
        grid=grid,
        in_specs=[
            pl.BlockSpec((_BB, _N, 16), lambda i: (i, 0, 0)),
            pl.BlockSpec((_BB, 4, _N, _N), lambda i: (i, 0, 0, 0)),
            pl.BlockSpec((16, 512), full),
            pl.BlockSpec((128, 512), full),
            pl.BlockSpec((128, 512), full),
            pl.BlockSpec((128, 128), full),
            pl.BlockSpec((128, 32), full),
            pl.BlockSpec((1, 32), full),
            pl.BlockSpec((128 * _BB, 8 * _BB), full),
            pl.BlockSpec((1, 8 * _BB), full),
            pl.BlockSpec((_N * _N, _ES), full),
            pl.BlockSpec((_ES, _N), full),
        ],
        out_specs=[
            pl.BlockSpec((_BB, _N, 16), lambda i: (i, 0, 0)),
            pl.BlockSpec((1, _E, 4 * _BB), lambda i: (i, 0, 0)),
        ],
        out_shape=[
            jax.ShapeDtypeStruct((b, _N, 16), jnp.float32),
            jax.ShapeDtypeStruct((b // _BB, _E, 4 * _BB), jnp.float32),
        ],
        compiler_params=pltpu.CompilerParams(
            dimension_semantics=("parallel",)),
        interpret=interpret,
    )(x, adj, w0c, w1c, w2c, wp, wn, bn, we, becat, bs, pp)
    return zn, ze


def kernel(inp_node_features, inp_adj_features, W0, W1, W2, Wproj,
           Wst_node, bst_node, Wst_edge, bst_edge):
    b = inp_node_features.shape[0]
    w0c = jnp.concatenate([W0[r] for r in range(4)], axis=1)
    w1c = jnp.concatenate([W1[r] for r in range(4)], axis=1)
    w2c = jnp.concatenate([W2[r] for r in range(4)], axis=1)
    wblk = jnp.zeros((128 * _BB, 8 * _BB), jnp.float32)
    for bb in range(_BB):
        wblk = wblk.at[128 * bb:128 * (bb + 1), 4 * bb:4 * (bb + 1)].set(
            Wst_edge[:, :4])
        wblk = wblk.at[128 * bb:128 * (bb + 1),
                       4 * _BB + 4 * bb:4 * _BB + 4 * (bb + 1)].set(
            Wst_edge[:, 4:])
    becat = jnp.concatenate([jnp.tile(bst_edge[:4], _BB),
                             jnp.tile(bst_edge[4:], _BB)]).reshape(1, 8 * _BB)
    zn, zew = _flow(inp_node_features, inp_adj_features, w0c, w1c,
                    w2c, Wproj, Wst_node, bst_node.reshape(1, 32), wblk,
                    becat, jnp.asarray(_BANDSEL), jnp.asarray(_PPAIR))
    ze = jnp.transpose(zew.reshape(b // _BB, _E, _BB, 4), (0, 2, 1, 3))
    return jnp.concatenate([zn.reshape(b, -1), ze.reshape(b, -1)], axis=1)


# speedup vs baseline: 1.3236x; 1.0132x over previous
import numpy as np
import jax
import jax.numpy as jnp
from jax.experimental import pallas as pl
from jax.experimental.pallas import tpu as pltpu

_N = 64
_EU = 12
_E = 690
_ES = 768
_BB = 32


def _edge_tables():
    src, dst = [], []
    for i in range(_N):
        for j in range(max(0, i - _EU), i):
            src.append(j)
            dst.append(i)
    src = np.asarray(src, np.int32)
    dst = np.asarray(dst, np.int32)
    bandsel = np.zeros((_N * _N, _ES), np.float32)
    bandsel[dst * _N + src, np.arange(_E)] = 1.0
    ppair = np.zeros((_ES, _N), np.float32)
    ppair[np.arange(_E), dst] = 1.0
    ppair[np.arange(_E), src] += 1.0
    return bandsel, ppair


_BANDSEL, _PPAIR = _edge_tables()


def _body(x_ref, adj_ref, w0_ref, w1_ref, w2_ref, wp_ref, wn_ref,
          bn_ref, we_ref, be_ref, bs_ref, pp_ref, zn_ref, ze_ref):
    f32 = jnp.float32
    bf16 = jnp.bfloat16
    x2d = x_ref[...].reshape(_BB * _N, 16)
    adj = adj_ref[...]

    h = x2d
    for wref in (w0_ref, w1_ref, w2_ref):
        mm = jnp.dot(h.astype(bf16), wref[...],
                     preferred_element_type=f32).astype(bf16)
        accs = []
        for b in range(_BB):
            adj_cat = jnp.concatenate([adj[b, r] for r in range(4)], axis=1)
            m_cat = jnp.concatenate(
                [mm[b * _N:(b + 1) * _N, 128 * r:128 * (r + 1)] for r in range(4)],
                axis=0)
            accs.append(jnp.dot(adj_cat, m_cat, preferred_element_type=f32))
        h = jax.nn.relu(jnp.concatenate(accs, axis=0))

    h = jnp.tanh(jnp.dot(h, wp_ref[...], preferred_element_type=f32))

    stn = jnp.dot(h, wn_ref[...], preferred_element_type=f32) + bn_ref[...]
    zn2d = x2d * jax.nn.sigmoid(stn[:, :16] + 2.0) + stn[:, 16:32]
    zn_ref[...] = zn2d.reshape(_BB, _N, 16)

    h_cat = jnp.concatenate([h[b * _N:(b + 1) * _N, :] for b in range(_BB)],
                            axis=1)
    he_sel = jnp.dot(h_cat, we_ref[...], preferred_element_type=f32)
    st = jnp.dot(pp_ref[...], he_sel, preferred_element_type=f32) + be_ref[...]

    adj_flat = adj.reshape(_BB * 4, _N, _N).reshape(_BB * 4, _N * _N)
    band_t = jnp.dot(adj_flat, bs_ref[...], preferred_element_type=f32)
    band = jnp.transpose(band_t)

    ze = band * jax.nn.sigmoid(st[:, :4 * _BB] + 2.0) + st[:, 4 * _BB:]
    ze_ref[0] = ze[:_E, :]


def _flow(x, adj, w0c, w1c, w2c, wp, wn, bn, we, becat, bs, pp,
          *, interpret=False):
    b = x.shape[0]
    grid = (b // _BB,)
    full = lambda i: (0, 0)
    zn, ze = pl.pallas_call(
        _body,
        grid=grid,
        in_specs=[
            pl.BlockSpec((_BB, _N, 16), lambda i: (i, 0, 0)),
            pl.BlockSpec((_BB, 4, _N, _N), lambda i: (i, 0, 0, 0)),
            pl.BlockSpec((16, 512), full),
            pl.BlockSpec((128, 512), full),
            pl.BlockSpec((128, 512), full),
            pl.BlockSpec((128, 128), full),
            pl.BlockSpec((128, 32), full),
            pl.BlockSpec((1, 32), full),
            pl.BlockSpec((128 * _BB, 8 * _BB), full),
            pl.BlockSpec((1, 8 * _BB), full),
            pl.BlockSpec((_N * _N, _ES), full),
            pl.BlockSpec((_ES, _N), full),
        ],
        out_specs=[
            pl.BlockSpec((_BB, _N, 16), lambda i: (i, 0, 0)),
            pl.BlockSpec((1, _E, 4 * _BB), lambda i: (i, 0, 0)),
        ],
        out_shape=[
            jax.ShapeDtypeStruct((b, _N, 16), jnp.float32),
            jax.ShapeDtypeStruct((b // _BB, _E, 4 * _BB), jnp.float32),
        ],
        compiler_params=pltpu.CompilerParams(
            dimension_semantics=("parallel",)),
        interpret=interpret,
    )(x, adj, w0c, w1c, w2c, wp, wn, bn, we, becat, bs, pp)
    return zn, ze


def kernel(inp_node_features, inp_adj_features, W0, W1, W2, Wproj,
           Wst_node, bst_node, Wst_edge, bst_edge):
    b = inp_node_features.shape[0]
    bf16 = jnp.bfloat16
    w0c = jnp.concatenate([W0[r] for r in range(4)], axis=1).astype(bf16)
    w1c = jnp.concatenate([W1[r] for r in range(4)], axis=1).astype(bf16)
    w2c = jnp.concatenate([W2[r] for r in range(4)], axis=1).astype(bf16)
    wblk = jnp.zeros((128 * _BB, 8 * _BB), jnp.float32)
    for bb in range(_BB):
        wblk = wblk.at[128 * bb:128 * (bb + 1), 4 * bb:4 * (bb + 1)].set(
            Wst_edge[:, :4])
        wblk = wblk.at[128 * bb:128 * (bb + 1),
                       4 * _BB + 4 * bb:4 * _BB + 4 * (bb + 1)].set(
            Wst_edge[:, 4:])
    becat = jnp.concatenate([jnp.tile(bst_edge[:4], _BB),
                             jnp.tile(bst_edge[4:], _BB)]).reshape(1, 8 * _BB)
    adjb = inp_adj_features.astype(bf16)
    zn, zew = _flow(inp_node_features, adjb, w0c, w1c,
                    w2c, Wproj, Wst_node, bst_node.reshape(1, 32), wblk,
                    becat, jnp.asarray(_BANDSEL).astype(bf16),
                    jnp.asarray(_PPAIR))
    ze = jnp.transpose(zew.reshape(b // _BB, _E, _BB, 4), (0, 2, 1, 3))
    return jnp.concatenate([zn.reshape(b, -1), ze.reshape(b, -1)], axis=1)
